# Initial kernel scaffold; baseline (speedup 1.0000x reference)
import jax
import jax.numpy as jnp
from jax.experimental import pallas as pl


def kernel(encoder_word, table, W, b):
    raise NotImplementedError("write your pallas kernel here")


# speedup vs baseline: 1.0400x; 1.0400x over previous
import functools

import jax
import jax.numpy as jnp
from jax import lax
from jax.experimental import pallas as pl
from jax.experimental.pallas import tpu as pltpu
from jax.experimental.pallas import tpu_sc as plsc


def _proj_body(t_ref, wt_ref, b_ref, o_ref):
    o_ref[...] = (
        jnp.dot(t_ref[...], wt_ref[...], preferred_element_type=jnp.float32)
        + b_ref[...]
    )


def _project_table(table, Wt, b_row, block_rows=8000):
    V, D = table.shape
    H = Wt.shape[1]
    grid = (pl.cdiv(V, block_rows),)
    return pl.pallas_call(
        _proj_body,
        grid=grid,
        in_specs=[
            pl.BlockSpec((block_rows, D), lambda i: (i, 0)),
            pl.BlockSpec((D, H), lambda i: (0, 0)),
            pl.BlockSpec((1, H), lambda i: (0, 0)),
        ],
        out_specs=pl.BlockSpec((block_rows, H), lambda i: (i, 0)),
        out_shape=jax.ShapeDtypeStruct((V, H), jnp.float32),
    )(table, Wt, b_row)


def _make_gather(N, H, n_workers, chunk):
    b_per_w = N // n_workers
    n_chunks = b_per_w // chunk
    mesh = plsc.VectorSubcoreMesh(core_axis_name="c", subcore_axis_name="s")

    @functools.partial(
        pl.kernel,
        mesh=mesh,
        out_type=jax.ShapeDtypeStruct((N, H), jnp.float32),
        scratch_types=[
            pltpu.VMEM((b_per_w,), jnp.int32),
            pltpu.VMEM((chunk, H), jnp.float32),
            pltpu.SemaphoreType.DMA,
        ],
        compiler_params=pltpu.CompilerParams(use_tc_tiling_on_sc=False),
    )
    def gather_k(idx_hbm, p_hbm, out_hbm, idx_v, rows_v, sem):
        nc = lax.axis_size("c")
        wid = lax.axis_index("s") * nc + lax.axis_index("c")
        base = wid * b_per_w
        pltpu.sync_copy(idx_hbm.at[pl.ds(base, b_per_w)], idx_v)

        def body(c, carry):
            off = c * chunk
            pltpu.async_copy(
                p_hbm.at[idx_v.at[pl.ds(off, chunk)]], rows_v, sem
            ).wait()
            pltpu.sync_copy(rows_v, out_hbm.at[pl.ds(base + off, chunk)])
            return carry

        lax.fori_loop(0, n_chunks, body, 0)

    return gather_k


def kernel(encoder_word, table, W, b):
    B, L = encoder_word.shape
    V, D = table.shape
    H = W.shape[0]
    N = B * L

    Wt = W.T
    b_row = b.reshape(1, H)
    idx = encoder_word.reshape(N).astype(jnp.int32)

    P = _project_table(table, Wt, b_row)

    info = plsc.get_sparse_core_info()
    n_workers = info.num_cores * info.num_subcores
    gather_k = _make_gather(N, H, n_workers, chunk=1024)
    out = gather_k(idx, P)
    return out.reshape(B, L, H)


# speedup vs baseline: 1.1857x; 1.1401x over previous
import functools

import jax
import jax.numpy as jnp
from jax import lax
from jax.experimental import pallas as pl
from jax.experimental.pallas import tpu as pltpu
from jax.experimental.pallas import tpu_sc as plsc


def _make_proj_manual(R, KD, NH, chunk, nbuf):
    ns = R // chunk

    def body(t_hbm, w_ref, b_ref, o_hbm, bin_ref, bout_ref, sin, sout):
        for i in range(nbuf):
            pltpu.make_async_copy(
                t_hbm.at[pl.ds(i * chunk, chunk)], bin_ref.at[i], sin.at[i]
            ).start()

        def step(s, carry):
            slot = lax.rem(s, nbuf)
            pltpu.make_async_copy(
                t_hbm.at[pl.ds(s * chunk, chunk)], bin_ref.at[slot], sin.at[slot]
            ).wait()

            @pl.when(s >= nbuf)
            def _():
                pltpu.make_async_copy(
                    bout_ref.at[slot],
                    o_hbm.at[pl.ds((s - nbuf) * chunk, chunk)],
                    sout.at[slot],
                ).wait()

            bout_ref[slot] = (
                jnp.dot(bin_ref[slot], w_ref[...],
                        preferred_element_type=jnp.float32)
                + b_ref[...]
            )
            pltpu.make_async_copy(
                bout_ref.at[slot], o_hbm.at[pl.ds(s * chunk, chunk)], sout.at[slot]
            ).start()

            @pl.when(s + nbuf < ns)
            def _():
                pltpu.make_async_copy(
                    t_hbm.at[pl.ds((s + nbuf) * chunk, chunk)],
                    bin_ref.at[slot],
                    sin.at[slot],
                ).start()

            return carry

        lax.fori_loop(0, ns, step, 0)

        def drain(k, carry):
            s = ns - nbuf + k
            slot = lax.rem(s, nbuf)
            pltpu.make_async_copy(
                bout_ref.at[slot], o_hbm.at[pl.ds(s * chunk, chunk)], sout.at[slot]
            ).wait()
            return carry

        lax.fori_loop(0, nbuf, drain, 0)

    return pl.pallas_call(
        body,
        in_specs=[
            pl.BlockSpec(memory_space=pltpu.MemorySpace.HBM),
            pl.BlockSpec(memory_space=pltpu.MemorySpace.VMEM),
            pl.BlockSpec(memory_space=pltpu.MemorySpace.VMEM),
        ],
        out_specs=pl.BlockSpec(memory_space=pltpu.MemorySpace.HBM),
        out_shape=jax.ShapeDtypeStruct((R, NH), jnp.float32),
        scratch_shapes=[
            pltpu.VMEM((nbuf, chunk, KD), jnp.float32),
            pltpu.VMEM((nbuf, chunk, NH), jnp.float32),
            pltpu.SemaphoreType.DMA((nbuf,)),
            pltpu.SemaphoreType.DMA((nbuf,)),
        ],
    )


def _make_gather(N, H, n_workers, chunk):
    b_per_w = N // n_workers
    n_chunks = b_per_w // chunk
    mesh = plsc.VectorSubcoreMesh(core_axis_name="c", subcore_axis_name="s")

    @functools.partial(
        pl.kernel,
        mesh=mesh,
        out_type=jax.ShapeDtypeStruct((N, H), jnp.float32),
        scratch_types=[
            pltpu.VMEM((b_per_w,), jnp.int32),
            pltpu.VMEM((chunk, H), jnp.float32),
            pltpu.SemaphoreType.DMA,
        ],
        compiler_params=pltpu.CompilerParams(use_tc_tiling_on_sc=False),
    )
    def gather_k(idx_hbm, p_hbm, out_hbm, idx_v, rows_v, sem):
        nc = lax.axis_size("c")
        wid = lax.axis_index("s") * nc + lax.axis_index("c")
        base = wid * b_per_w
        pltpu.sync_copy(idx_hbm.at[pl.ds(base, b_per_w)], idx_v)

        def body(c, carry):
            off = c * chunk
            pltpu.async_copy(
                p_hbm.at[idx_v.at[pl.ds(off, chunk)]], rows_v, sem
            ).wait()
            pltpu.sync_copy(rows_v, out_hbm.at[pl.ds(base + off, chunk)])
            return carry

        lax.fori_loop(0, n_chunks, body, 0)

    return gather_k


def kernel(encoder_word, table, W, b):
    B, L = encoder_word.shape
    V, D = table.shape
    H = W.shape[0]
    N = B * L

    import jax.scipy.linalg as jsl
    Wt = W.T
    W4t = jsl.block_diag(Wt, Wt, Wt, Wt)
    b4 = jnp.tile(b, 4).reshape(1, 4 * H)
    table4 = table.reshape(V // 4, 4 * D)
    idx = encoder_word.reshape(N).astype(jnp.int32)

    proj = _make_proj_manual(V // 4, 4 * D, 4 * H, chunk=2000, nbuf=8)
    P_wide = proj(table4, W4t, b4)
    P = P_wide.reshape(V, H)

    info = plsc.get_sparse_core_info()
    n_workers = info.num_cores * info.num_subcores
    gather_k = _make_gather(N, H, n_workers, chunk=1024)
    out = gather_k(idx, P)
    return out.reshape(B, L, H)


# speedup vs baseline: 1.2516x; 1.0555x over previous
import functools

import jax
import jax.numpy as jnp
from jax import lax
from jax.experimental import pallas as pl
from jax.experimental.pallas import tpu as pltpu
from jax.experimental.pallas import tpu_sc as plsc


def _make_proj_manual(V, D, NH, chunk, nbuf):
    ns = V // chunk

    def body(t_hbm, w_ref, b_ref, o_hbm, bin_ref, bout_ref, sin, sout):
        for i in range(nbuf):
            pltpu.make_async_copy(
                t_hbm.at[pl.ds(i * chunk, chunk)], bin_ref.at[i], sin.at[i]
            ).start()

        def step(s, carry):
            slot = lax.rem(s, nbuf)
            pltpu.make_async_copy(
                t_hbm.at[pl.ds(s * chunk, chunk)], bin_ref.at[slot], sin.at[slot]
            ).wait()

            @pl.when(s >= nbuf)
            def _():
                pltpu.make_async_copy(
                    bout_ref.at[slot],
                    o_hbm.at[pl.ds((s - nbuf) * chunk, chunk)],
                    sout.at[slot],
                ).wait()

            bout_ref[slot] = (
                jnp.dot(bin_ref[slot], w_ref[...],
                        preferred_element_type=jnp.float32)
                + b_ref[...]
            )
            pltpu.make_async_copy(
                bout_ref.at[slot], o_hbm.at[pl.ds(s * chunk, chunk)], sout.at[slot]
            ).start()

            @pl.when(s + nbuf < ns)
            def _():
                pltpu.make_async_copy(
                    t_hbm.at[pl.ds((s + nbuf) * chunk, chunk)],
                    bin_ref.at[slot],
                    sin.at[slot],
                ).start()

            return carry

        lax.fori_loop(0, ns, step, 0)

        def drain(k, carry):
            s = ns - nbuf + k
            slot = lax.rem(s, nbuf)
            pltpu.make_async_copy(
                bout_ref.at[slot], o_hbm.at[pl.ds(s * chunk, chunk)], sout.at[slot]
            ).wait()
            return carry

        lax.fori_loop(0, nbuf, drain, 0)

    return pl.pallas_call(
        body,
        in_specs=[
            pl.BlockSpec(memory_space=pltpu.MemorySpace.HBM),
            pl.BlockSpec(memory_space=pltpu.MemorySpace.VMEM),
            pl.BlockSpec(memory_space=pltpu.MemorySpace.VMEM),
        ],
        out_specs=pl.BlockSpec(memory_space=pltpu.MemorySpace.HBM),
        out_shape=jax.ShapeDtypeStruct((V, NH), jnp.float32),
        scratch_shapes=[
            pltpu.VMEM((nbuf, chunk, D), jnp.float32),
            pltpu.VMEM((nbuf, chunk, NH), jnp.float32),
            pltpu.SemaphoreType.DMA((nbuf,)),
            pltpu.SemaphoreType.DMA((nbuf,)),
        ],
    )


def _make_gather(N, NH, n_workers, chunk):
    b_per_w = N // n_workers
    n_chunks = b_per_w // chunk
    mesh = plsc.VectorSubcoreMesh(core_axis_name="c", subcore_axis_name="s")

    @functools.partial(
        pl.kernel,
        mesh=mesh,
        out_type=jax.ShapeDtypeStruct((N, NH), jnp.float32),
        scratch_types=[
            pltpu.VMEM((b_per_w,), jnp.int32),
            pltpu.VMEM((chunk, NH), jnp.float32),
            pltpu.SemaphoreType.DMA,
        ],
    )
    def gather_k(idx_hbm, p_hbm, out_hbm, idx_v, rows_v, sem):
        nc = lax.axis_size("c")
        wid = lax.axis_index("s") * nc + lax.axis_index("c")
        base = wid * b_per_w
        pltpu.sync_copy(idx_hbm.at[pl.ds(base, b_per_w)], idx_v)

        def body(c, carry):
            off = c * chunk
            pltpu.async_copy(
                p_hbm.at[idx_v.at[pl.ds(off, chunk)]], rows_v, sem
            ).wait()
            pltpu.sync_copy(rows_v, out_hbm.at[pl.ds(base + off, chunk)])
            return carry

        lax.fori_loop(0, n_chunks, body, 0)

    return gather_k


def kernel(encoder_word, table, W, b):
    B, L = encoder_word.shape
    V, D = table.shape
    H = W.shape[0]
    N = B * L
    NH = 128

    Wp = jnp.zeros((D, NH), jnp.float32).at[:, :H].set(W.T)
    bp = jnp.zeros((1, NH), jnp.float32).at[:, :H].set(b)
    idx = encoder_word.reshape(N).astype(jnp.int32)

    proj = _make_proj_manual(V, D, NH, chunk=4000, nbuf=6)
    P_pad = proj(table, Wp, bp)

    info = plsc.get_sparse_core_info()
    n_workers = info.num_cores * info.num_subcores
    gather_k = _make_gather(N, NH, n_workers, chunk=512)
    out_pad = gather_k(idx, P_pad)
    return out_pad[:, :H].reshape(B, L, H)


# speedup vs baseline: 1.2682x; 1.0133x over previous
import functools

import jax
import jax.numpy as jnp
from jax import lax
from jax.experimental import pallas as pl
from jax.experimental.pallas import tpu as pltpu
from jax.experimental.pallas import tpu_sc as plsc


def _make_proj_manual(V, D, NH, chunk, nbuf):
    ns = V // chunk

    def body(t_hbm, w_ref, b_ref, o_hbm, bin_ref, bout_ref, sin, sout):
        for i in range(nbuf):
            pltpu.make_async_copy(
                t_hbm.at[pl.ds(i * chunk, chunk)], bin_ref.at[i], sin.at[i]
            ).start()

        def step(s, carry):
            slot = lax.rem(s, nbuf)
            pltpu.make_async_copy(
                t_hbm.at[pl.ds(s * chunk, chunk)], bin_ref.at[slot], sin.at[slot]
            ).wait()

            @pl.when(s >= nbuf)
            def _():
                pltpu.make_async_copy(
                    bout_ref.at[slot],
                    o_hbm.at[pl.ds((s - nbuf) * chunk, chunk)],
                    sout.at[slot],
                ).wait()

            bout_ref[slot] = (
                jnp.dot(bin_ref[slot], w_ref[...],
                        preferred_element_type=jnp.float32)
                + b_ref[...]
            )
            pltpu.make_async_copy(
                bout_ref.at[slot], o_hbm.at[pl.ds(s * chunk, chunk)], sout.at[slot]
            ).start()

            @pl.when(s + nbuf < ns)
            def _():
                pltpu.make_async_copy(
                    t_hbm.at[pl.ds((s + nbuf) * chunk, chunk)],
                    bin_ref.at[slot],
                    sin.at[slot],
                ).start()

            return carry

        lax.fori_loop(0, ns, step, 0)

        def drain(k, carry):
            s = ns - nbuf + k
            slot = lax.rem(s, nbuf)
            pltpu.make_async_copy(
                bout_ref.at[slot], o_hbm.at[pl.ds(s * chunk, chunk)], sout.at[slot]
            ).wait()
            return carry

        lax.fori_loop(0, nbuf, drain, 0)

    return pl.pallas_call(
        body,
        in_specs=[
            pl.BlockSpec(memory_space=pltpu.MemorySpace.HBM),
            pl.BlockSpec(memory_space=pltpu.MemorySpace.VMEM),
            pl.BlockSpec(memory_space=pltpu.MemorySpace.VMEM),
        ],
        out_specs=pl.BlockSpec(memory_space=pltpu.MemorySpace.HBM),
        out_shape=jax.ShapeDtypeStruct((V, NH), jnp.float32),
        scratch_shapes=[
            pltpu.VMEM((nbuf, chunk, D), jnp.float32),
            pltpu.VMEM((nbuf, chunk, NH), jnp.float32),
            pltpu.SemaphoreType.DMA((nbuf,)),
            pltpu.SemaphoreType.DMA((nbuf,)),
        ],
    )


def _make_gather(B, L, NH, n_workers, chunk):
    N = B * L
    b_rows = B // n_workers
    b_per_w = N // n_workers
    n_super = b_per_w // (2 * chunk)
    mesh = plsc.VectorSubcoreMesh(core_axis_name="c", subcore_axis_name="s")

    @functools.partial(
        pl.kernel,
        mesh=mesh,
        out_type=jax.ShapeDtypeStruct((N, NH), jnp.float32),
        scratch_types=[
            pltpu.VMEM((b_per_w,), jnp.int32),
            pltpu.VMEM((2, chunk, NH), jnp.float32),
            pltpu.SemaphoreType.DMA((2,)),
        ],
    )
    def gather_k(idx_hbm, p_hbm, out_hbm, idx_v, rows_v, sem):
        nc = lax.axis_size("c")
        wid = lax.axis_index("s") * nc + lax.axis_index("c")
        base = wid * b_per_w
        pltpu.sync_copy(idx_hbm.at[pl.ds(base, b_per_w)], idx_v)

        def start_gather(c, slot):
            pltpu.async_copy(
                p_hbm.at[idx_v.at[pl.ds(c * chunk, chunk)]],
                rows_v.at[slot],
                sem.at[slot],
            )

        def wait_gather(slot):
            pltpu.make_async_copy(
                p_hbm.at[pl.ds(0, chunk)], rows_v.at[slot], sem.at[slot]
            ).wait()

        start_gather(0, 0)

        def sbody(g, carry):
            c0 = 2 * g
            start_gather(c0 + 1, 1)
            wait_gather(0)
            pltpu.sync_copy(
                rows_v.at[0], out_hbm.at[pl.ds(base + c0 * chunk, chunk)]
            )

            @pl.when(g + 1 < n_super)
            def _():
                start_gather(c0 + 2, 0)

            wait_gather(1)
            pltpu.sync_copy(
                rows_v.at[1], out_hbm.at[pl.ds(base + (c0 + 1) * chunk, chunk)]
            )
            return carry

        lax.fori_loop(0, n_super, sbody, 0)

    return gather_k


def kernel(encoder_word, table, W, b):
    B, L = encoder_word.shape
    V, D = table.shape
    H = W.shape[0]
    N = B * L
    NH = 128

    Wp = jnp.zeros((D, NH), jnp.float32).at[:, :H].set(W.T)
    bp = jnp.zeros((1, NH), jnp.float32).at[:, :H].set(b)

    proj = _make_proj_manual(V, D, NH, chunk=2000, nbuf=12)
    P_pad = proj(table, Wp, bp)

    info = plsc.get_sparse_core_info()
    n_workers = info.num_cores * info.num_subcores
    gather_k = _make_gather(B, L, NH, n_workers, chunk=256)
    idx = encoder_word.reshape(N).astype(jnp.int32)
    out_pad = gather_k(idx, P_pad)
    return out_pad[:, :H].reshape(B, L, H)
